# baseline (speedup 1.0000x reference)
import jax
import jax.numpy as jnp
from jax import lax
from jax.experimental import pallas as pl
from jax.experimental.pallas import tpu as pltpu, tpu_sc as plsc

EMBED_SIZE = 64
BATCH = 16384

_info = plsc.get_sparse_core_info()
_NC, _NS = _info.num_cores, _info.num_subcores
_NW = _NC * _NS
_B_PER_W = BATCH // _NW


def _gather_body(table_hbm, idx_hbm, out_hbm, idx_v, rows_v, sem):
    wid = lax.axis_index("s") * _NC + lax.axis_index("c")
    base = wid * _B_PER_W
    pltpu.sync_copy(idx_hbm.at[pl.ds(base, _B_PER_W)], idx_v)
    pltpu.async_copy(table_hbm.at[idx_v], rows_v, sem).wait()
    pltpu.sync_copy(rows_v, out_hbm.at[pl.ds(base, _B_PER_W)])


@jax.jit
def _lookup(embedding, inputs):
    mesh = plsc.VectorSubcoreMesh(core_axis_name="c", subcore_axis_name="s")
    k = pl.kernel(
        _gather_body,
        mesh=mesh,
        out_type=jax.ShapeDtypeStruct((BATCH, EMBED_SIZE), jnp.float32),
        scratch_types=[
            pltpu.VMEM((_B_PER_W,), jnp.int32),
            pltpu.VMEM((_B_PER_W, EMBED_SIZE), jnp.float32),
            pltpu.SemaphoreType.DMA,
        ],
        compiler_params=pltpu.CompilerParams(use_tc_tiling_on_sc=False),
    )
    return k(embedding, inputs)


def kernel(inputs, embedding):
    return _lookup(embedding, inputs.astype(jnp.int32))


# speedup vs baseline: 1.1282x; 1.1282x over previous
import jax
import jax.numpy as jnp
from jax import lax
from jax.experimental import pallas as pl
from jax.experimental.pallas import tpu as pltpu, tpu_sc as plsc

NODE = 1000000
D = 64
B = 16384

_info = plsc.get_sparse_core_info()
_NC, _NS = _info.num_cores, _info.num_subcores
_NW = _NC * _NS

_NTILECOL = (NODE + 127) // 128
_TAIL_COL = (NODE // 128) * 128
_GCOLS = 512
_REGION = B + 16
_DUMMY = B
_OUT_ROWS = B + 8
_SENT = 0x7FFFFFF0

_IOTA = lambda: lax.iota(jnp.int32, 16)


def _splat(x):
    return jnp.full((16,), x, jnp.int32)


def _k1_body(table, idx_hbm, vals, jlist, counts, idxbuf, li, lj, stagedj,
             pend, group, tailbuf, stage, cnt16, sem):
    wid = lax.axis_index("s") * _NC + lax.axis_index("c")
    clo = wid * _NTILECOL // _NW
    chi = (wid + 1) * _NTILECOL // _NW
    slo = clo * 128
    shi = chi * 128
    send = jnp.minimum(shi, _TAIL_COL)
    ngroups = (send - slo + _GCOLS - 1) // _GCOLS
    rbase = wid * _REGION

    pltpu.sync_copy(idx_hbm, idxbuf)

    def init_j(v, c):
        plsc.store_scatter(stagedj, [_IOTA() + v * 16], _splat(_DUMMY))
        return c
    lax.fori_loop(0, _REGION // 16, init_j, 0)

    def part(v, nw):
        iv = idxbuf[pl.ds(v * 16, 16)]
        m = (iv >= slo) & (iv < shi)
        rank = plsc.cumsum(m.astype(jnp.int32)) - 1
        plsc.store_scatter(li, [nw + rank], iv, mask=m)
        plsc.store_scatter(lj, [nw + rank], _IOTA() + v * 16, mask=m)
        return nw + plsc.all_reduce_population_count(m)[0]
    nw = lax.fori_loop(0, B // 16, part, jnp.int32(0))
    plsc.store_scatter(li, [nw + _IOTA()], jnp.broadcast_to(_SENT, (16,)))
    nvec = (nw + 15) // 16

    def scan_block(src_ref, s_lo, s_hi, ptot):
        def sc(v, np_):
            iv = li[pl.ds(v * 16, 16)]
            jv = lj[pl.ds(v * 16, 16)]
            m = (iv >= s_lo) & (iv < s_hi)
            rank = plsc.cumsum(m.astype(jnp.int32)) - 1
            plsc.store_scatter(pend, [np_ + rank], iv - s_lo, mask=m)
            plsc.store_scatter(stagedj, [ptot + np_ + rank], jv, mask=m)
            return np_ + plsc.all_reduce_population_count(m)[0]
        npend = lax.fori_loop(0, nvec, sc, jnp.int32(0))

        def flush(b, c):
            ccv = plsc.load_gather(pend, [b * 16 + _IOTA()])
            sm = _IOTA() < (npend - b * 16)
            for d in range(D):
                v = plsc.load_gather(src_ref, [_splat(d), ccv], mask=sm)
                plsc.store_scatter(stage, [_IOTA() * D + d], v, mask=sm)
            p0 = ptot + b * 16
            pltpu.sync_copy(stage, vals.at[pl.ds((rbase + p0) * D, 16 * D)])
            return c
        lax.fori_loop(0, (npend + 15) // 16, flush, 0)
        return ptot + npend

    def per_group(g, ptot):
        s = slo + g * _GCOLS
        pltpu.sync_copy(table.at[:, pl.ds(s, _GCOLS)], group)
        return scan_block(group, s, jnp.minimum(s + _GCOLS, send), ptot)
    ptot = lax.fori_loop(0, ngroups, per_group, jnp.int32(0))

    pltpu.sync_copy(table.at[:, pl.ds(_TAIL_COL, NODE - _TAIL_COL)], tailbuf)
    ptot = scan_block(tailbuf, jnp.int32(_TAIL_COL), jnp.int32(NODE + 128), ptot)

    cnt16[...] = jnp.broadcast_to(ptot, (16,))
    pltpu.sync_copy(cnt16, counts.at[pl.ds(wid * 16, 16)])
    pltpu.sync_copy(stagedj, jlist.at[pl.ds(rbase, _REGION)])


def _k2_body(vals2d, jlist, counts, out, jbuf, rows, cbuf, sem):
    wid = lax.axis_index("s") * _NC + lax.axis_index("c")
    rbase = wid * _REGION
    pltpu.sync_copy(counts.at[pl.ds(wid * 16, 16)], cbuf)
    n = jnp.max(cbuf[...])
    nchunks = (n + _CHUNK - 1) // _CHUNK

    def chunk(c, carry):
        base = rbase + c * _CHUNK
        pltpu.sync_copy(jlist.at[pl.ds(base, _CHUNK)], jbuf)
        pltpu.sync_copy(vals2d.at[pl.ds(base, _CHUNK)], rows)
        pltpu.async_copy(rows, out.at[jbuf], sem).wait()
        return carry
    lax.fori_loop(0, nchunks, chunk, 0)


_CHUNK = 1024


@jax.jit
def _lookup(embedding, inputs):
    mesh = plsc.VectorSubcoreMesh(core_axis_name="c", subcore_axis_name="s")
    k1 = pl.kernel(
        _k1_body,
        mesh=mesh,
        out_type=(
            jax.ShapeDtypeStruct((_NW * _REGION * D,), jnp.float32),
            jax.ShapeDtypeStruct((_NW * _REGION,), jnp.int32),
            jax.ShapeDtypeStruct((_NW * 16,), jnp.int32),
        ),
        scratch_types=[
            pltpu.VMEM((B,), jnp.int32),
            pltpu.VMEM((_REGION,), jnp.int32),
            pltpu.VMEM((_REGION,), jnp.int32),
            pltpu.VMEM((_REGION,), jnp.int32),
            pltpu.VMEM((_REGION,), jnp.int32),
            pltpu.VMEM((D, _GCOLS), jnp.float32),
            pltpu.VMEM((D, NODE - _TAIL_COL), jnp.float32),
            pltpu.VMEM((16 * D,), jnp.float32),
            pltpu.VMEM((16,), jnp.int32),
            pltpu.SemaphoreType.DMA,
        ],
        compiler_params=pltpu.CompilerParams(
            use_tc_tiling_on_sc=True, needs_layout_passes=False),
    )
    vals, jlist, counts = k1(embedding.T, inputs)

    k2 = pl.kernel(
        _k2_body,
        mesh=mesh,
        out_type=jax.ShapeDtypeStruct((_OUT_ROWS, D), jnp.float32),
        scratch_types=[
            pltpu.VMEM((_CHUNK,), jnp.int32),
            pltpu.VMEM((_CHUNK, D), jnp.float32),
            pltpu.VMEM((16,), jnp.int32),
            pltpu.SemaphoreType.DMA,
        ],
        compiler_params=pltpu.CompilerParams(
            use_tc_tiling_on_sc=False, needs_layout_passes=False),
    )
    out = k2(vals.reshape(_NW * _REGION, D), jlist, counts)
    return out[:B]


def kernel(inputs, embedding):
    return _lookup(embedding, inputs.astype(jnp.int32))


# speedup vs baseline: 2.9151x; 2.5840x over previous
import jax
import jax.numpy as jnp
from jax import lax
from jax.experimental import pallas as pl
from jax.experimental.pallas import tpu as pltpu, tpu_sc as plsc

NODE = 1000000
D = 64
B = 16384

_info = plsc.get_sparse_core_info()
_NC, _NS = _info.num_cores, _info.num_subcores
_NW = _NC * _NS

_NTILECOL = (NODE + 127) // 128
_TAIL_COL = (NODE // 128) * 128
_GCOLS = 384
_REGION = B + 16
_DUMMY = B
_OUT_ROWS = B + 1024
_SENT = 0x7FFFFFF0

_IOTA = lambda: lax.iota(jnp.int32, 16)


def _splat(x):
    return jnp.full((16,), x, jnp.int32)


def _k1_body(table, idx_hbm, vals, jlist, counts, li, lj, stagedj,
             pend, group_a, group_b, tailbuf, stage, cnt16, sem):
    idxbuf = pend.at[pl.ds(0, B)]
    wid = lax.axis_index("s") * _NC + lax.axis_index("c")
    clo = wid * _NTILECOL // _NW
    chi = (wid + 1) * _NTILECOL // _NW
    slo = clo * 128
    shi = chi * 128
    send = jnp.minimum(shi, _TAIL_COL)
    ngroups = (send - slo + _GCOLS - 1) // _GCOLS
    rbase = wid * _REGION

    pltpu.sync_copy(idx_hbm, idxbuf)

    def init_j(v, c):
        pv = _IOTA() + v * 16
        plsc.store_scatter(stagedj, [pv], _splat(_DUMMY) + (pv & 1023))
        return c
    lax.fori_loop(0, _REGION // 16, init_j, 0)

    def part(v, nw):
        iv = idxbuf[pl.ds(v * 16, 16)]
        m = (iv >= slo) & (iv < shi)
        rank = plsc.cumsum(m.astype(jnp.int32)) - 1
        plsc.store_scatter(li, [nw + rank], iv, mask=m)
        plsc.store_scatter(lj, [nw + rank], _IOTA() + v * 16, mask=m)
        return nw + plsc.all_reduce_population_count(m)[0]
    nw = lax.fori_loop(0, B // 16, part, jnp.int32(0))
    plsc.store_scatter(li, [nw + _IOTA()], jnp.broadcast_to(_SENT, (16,)))
    nvec = (nw + 15) // 16

    def scan_block(src_ref, s_lo, s_hi, ptot, cc_base=None):
        cb = s_lo if cc_base is None else cc_base
        def sc(v, np_):
            iv = li[pl.ds(v * 16, 16)]
            jv = lj[pl.ds(v * 16, 16)]
            m = (iv >= s_lo) & (iv < s_hi)
            rank = plsc.cumsum(m.astype(jnp.int32)) - 1
            plsc.store_scatter(pend, [np_ + rank], iv - cb, mask=m)
            plsc.store_scatter(stagedj, [ptot + np_ + rank], jv, mask=m)
            return np_ + plsc.all_reduce_population_count(m)[0]
        npend = lax.fori_loop(0, nvec, sc, jnp.int32(0))

        def flush(b, c):
            ccv = plsc.load_gather(pend, [b * 16 + _IOTA()])
            sm = _IOTA() < (npend - b * 16)
            for d in range(D):
                v = plsc.load_gather(src_ref, [_splat(d), ccv], mask=sm)
                plsc.store_scatter(stage, [_IOTA() * D + d], v, mask=sm)
            p0 = ptot + b * 16
            pltpu.sync_copy(stage, vals.at[pl.ds((rbase + p0) * D, 16 * D)])
            return c
        lax.fori_loop(0, (npend + 15) // 16, flush, 0)
        return ptot + npend

    def fetch_base(g):
        return jnp.minimum(slo + g * _GCOLS, _TAIL_COL - _GCOLS)

    def start_fetch(g, buf):
        return pltpu.async_copy(
            table.at[:, pl.ds(fetch_base(g), _GCOLS)], buf, sem)

    def wait_fetch(g, buf):
        pltpu.make_async_copy(
            table.at[:, pl.ds(fetch_base(g), _GCOLS)], buf, sem).wait()

    def process(g, buf, ptot):
        s = slo + g * _GCOLS
        return scan_block(buf, s, jnp.minimum(s + _GCOLS, send), ptot,
                          cc_base=fetch_base(g))

    start_fetch(0, group_a)

    def per_pair(i, ptot):
        g0 = 2 * i
        g1 = 2 * i + 1
        wait_fetch(g0, group_a)

        @pl.when(g1 < ngroups)
        def _():
            start_fetch(g1, group_b)
        ptot = process(g0, group_a, ptot)

        def do_b(pt):
            wait_fetch(g1, group_b)

            @pl.when(g1 + 1 < ngroups)
            def _():
                start_fetch(g1 + 1, group_a)
            return process(g1, group_b, pt)
        return lax.cond(g1 < ngroups, do_b, lambda pt: pt, ptot)
    ptot = lax.fori_loop(0, (ngroups + 1) // 2, per_pair, jnp.int32(0))

    pltpu.sync_copy(table.at[:, pl.ds(_TAIL_COL, NODE - _TAIL_COL)], tailbuf)
    ptot = scan_block(tailbuf, jnp.int32(_TAIL_COL), jnp.int32(NODE + 128), ptot)

    cnt16[...] = jnp.broadcast_to(ptot, (16,))
    pltpu.sync_copy(cnt16, counts.at[pl.ds(wid * 16, 16)])
    pltpu.sync_copy(stagedj, jlist.at[pl.ds(rbase, _REGION)])


def _k2_body(vals2d, jlist, counts, out, jbuf, rows, cbuf, sem):
    wid = lax.axis_index("s") * _NC + lax.axis_index("c")
    rbase = wid * _REGION
    pltpu.sync_copy(counts.at[pl.ds(wid * 16, 16)], cbuf)
    n = jnp.max(cbuf[...])
    nchunks = (n + _CHUNK - 1) // _CHUNK

    def chunk(c, carry):
        base = rbase + c * _CHUNK
        pltpu.sync_copy(jlist.at[pl.ds(base, _CHUNK)], jbuf)
        pltpu.sync_copy(vals2d.at[pl.ds(base, _CHUNK)], rows)
        pltpu.async_copy(rows, out.at[jbuf], sem).wait()
        return carry
    lax.fori_loop(0, nchunks, chunk, 0)


_CHUNK = 1024


@jax.jit
def _lookup(embedding, inputs):
    mesh = plsc.VectorSubcoreMesh(core_axis_name="c", subcore_axis_name="s")
    k1 = pl.kernel(
        _k1_body,
        mesh=mesh,
        out_type=(
            jax.ShapeDtypeStruct((_NW * _REGION * D,), jnp.float32),
            jax.ShapeDtypeStruct((_NW * _REGION,), jnp.int32),
            jax.ShapeDtypeStruct((_NW * 16,), jnp.int32),
        ),
        scratch_types=[
            pltpu.VMEM((_REGION,), jnp.int32),
            pltpu.VMEM((_REGION,), jnp.int32),
            pltpu.VMEM((_REGION,), jnp.int32),
            pltpu.VMEM((_REGION,), jnp.int32),
            pltpu.VMEM((D, _GCOLS), jnp.float32),
            pltpu.VMEM((D, _GCOLS), jnp.float32),
            pltpu.VMEM((D, NODE - _TAIL_COL), jnp.float32),
            pltpu.VMEM((16 * D,), jnp.float32),
            pltpu.VMEM((16,), jnp.int32),
            pltpu.SemaphoreType.DMA,
        ],
        compiler_params=pltpu.CompilerParams(
            use_tc_tiling_on_sc=True, needs_layout_passes=False),
    )
    vals, jlist, counts = k1(embedding.T, inputs)

    k2 = pl.kernel(
        _k2_body,
        mesh=mesh,
        out_type=jax.ShapeDtypeStruct((_OUT_ROWS, D), jnp.float32),
        scratch_types=[
            pltpu.VMEM((_CHUNK,), jnp.int32),
            pltpu.VMEM((_CHUNK, D), jnp.float32),
            pltpu.VMEM((16,), jnp.int32),
            pltpu.SemaphoreType.DMA,
        ],
        compiler_params=pltpu.CompilerParams(
            use_tc_tiling_on_sc=False, needs_layout_passes=False),
    )
    out = k2(vals.reshape(_NW * _REGION, D), jlist, counts)
    return out[:B]


def kernel(inputs, embedding):
    return _lookup(embedding, inputs.astype(jnp.int32))


# speedup vs baseline: 2.9466x; 1.0108x over previous
import jax
import jax.numpy as jnp
from jax import lax
from jax.experimental import pallas as pl
from jax.experimental.pallas import tpu as pltpu, tpu_sc as plsc

NODE = 1000000
D = 64
B = 16384

_info = plsc.get_sparse_core_info()
_NC, _NS = _info.num_cores, _info.num_subcores
_NW = _NC * _NS

_NTILECOL = (NODE + 127) // 128
_TAIL_COL = (NODE // 128) * 128
_GCOLS = 384
_REGION = B + 16
_DUMMY = B
_OUT_ROWS = B + 1024
_SENT = 0x7FFFFFF0

_IOTA = lambda: lax.iota(jnp.int32, 16)


def _splat(x):
    return jnp.full((16,), x, jnp.int32)


def _k1_body(table, idx_hbm, vals, jlist, counts, li, lj, stagedj,
             pend, group_a, group_b, tailbuf, stage, cnt16, sem, sem2):
    idxbuf = pend.at[pl.ds(0, B)]
    wid = lax.axis_index("s") * _NC + lax.axis_index("c")
    clo = wid * _NTILECOL // _NW
    chi = (wid + 1) * _NTILECOL // _NW
    slo = clo * 128
    fetch_base0 = jnp.minimum(slo, _TAIL_COL - _GCOLS)
    shi = chi * 128
    send = jnp.minimum(shi, _TAIL_COL)
    ngroups = (send - slo + _GCOLS - 1) // _GCOLS
    rbase = wid * _REGION

    pltpu.sync_copy(idx_hbm, idxbuf)
    tail_cp = pltpu.async_copy(
        table.at[:, pl.ds(_TAIL_COL, NODE - _TAIL_COL)], tailbuf, sem2)

    pltpu.async_copy(table.at[:, pl.ds(fetch_base0, _GCOLS)], group_a, sem)

    def init_j(v, c):
        pv = _IOTA() + v * 16
        plsc.store_scatter(stagedj, [pv], _splat(_DUMMY) + (pv & 1023))
        return c
    lax.fori_loop(0, _REGION // 16, init_j, 0)

    def part(v, nw):
        iv = idxbuf[pl.ds(v * 16, 16)]
        m = (iv >= slo) & (iv < shi)
        rank = plsc.cumsum(m.astype(jnp.int32)) - 1
        plsc.store_scatter(li, [nw + rank], iv, mask=m)
        plsc.store_scatter(lj, [nw + rank], _IOTA() + v * 16, mask=m)
        return nw + plsc.all_reduce_population_count(m)[0]
    nw = lax.fori_loop(0, B // 16, part, jnp.int32(0))
    plsc.store_scatter(li, [nw + _IOTA()], jnp.broadcast_to(_SENT, (16,)))
    nvec = (nw + 15) // 16

    def scan_block(src_ref, s_lo, s_hi, ptot, cc_base=None):
        cb = s_lo if cc_base is None else cc_base
        def sc(v, np_):
            iv = li[pl.ds(v * 16, 16)]
            jv = lj[pl.ds(v * 16, 16)]
            m = (iv >= s_lo) & (iv < s_hi)
            rank = plsc.cumsum(m.astype(jnp.int32)) - 1
            plsc.store_scatter(pend, [np_ + rank], iv - cb, mask=m)
            plsc.store_scatter(stagedj, [ptot + np_ + rank], jv, mask=m)
            return np_ + plsc.all_reduce_population_count(m)[0]
        npend = lax.fori_loop(0, nvec, sc, jnp.int32(0))

        def flush(b, c):
            ccv = plsc.load_gather(pend, [b * 16 + _IOTA()])
            sm = _IOTA() < (npend - b * 16)
            for d in range(D):
                v = plsc.load_gather(src_ref, [_splat(d), ccv], mask=sm)
                plsc.store_scatter(stage, [_IOTA() * D + d], v, mask=sm)
            p0 = ptot + b * 16
            pltpu.sync_copy(stage, vals.at[pl.ds((rbase + p0) * D, 16 * D)])
            return c
        lax.fori_loop(0, (npend + 15) // 16, flush, 0)
        return ptot + npend

    def fetch_base(g):
        return jnp.minimum(slo + g * _GCOLS, _TAIL_COL - _GCOLS)

    def start_fetch(g, buf):
        return pltpu.async_copy(
            table.at[:, pl.ds(fetch_base(g), _GCOLS)], buf, sem)

    def wait_fetch(g, buf):
        pltpu.make_async_copy(
            table.at[:, pl.ds(fetch_base(g), _GCOLS)], buf, sem).wait()

    def process(g, buf, ptot):
        s = slo + g * _GCOLS
        return scan_block(buf, s, jnp.minimum(s + _GCOLS, send), ptot,
                          cc_base=fetch_base(g))

    def per_pair(i, ptot):
        g0 = 2 * i
        g1 = 2 * i + 1
        wait_fetch(g0, group_a)

        @pl.when(g1 < ngroups)
        def _():
            start_fetch(g1, group_b)
        ptot = process(g0, group_a, ptot)

        def do_b(pt):
            wait_fetch(g1, group_b)

            @pl.when(g1 + 1 < ngroups)
            def _():
                start_fetch(g1 + 1, group_a)
            return process(g1, group_b, pt)
        return lax.cond(g1 < ngroups, do_b, lambda pt: pt, ptot)
    ptot = lax.fori_loop(0, (ngroups + 1) // 2, per_pair, jnp.int32(0))

    tail_cp.wait()
    ptot = scan_block(tailbuf, jnp.int32(_TAIL_COL), jnp.int32(NODE + 128), ptot)

    cnt16[...] = jnp.broadcast_to(ptot, (16,))
    pltpu.sync_copy(cnt16, counts.at[pl.ds(wid * 16, 16)])
    pltpu.sync_copy(stagedj, jlist.at[pl.ds(rbase, _REGION)])


def _k2_body(vals2d, jlist, counts, out, jbuf, rows, cbuf, sem):
    wid = lax.axis_index("s") * _NC + lax.axis_index("c")
    rbase = wid * _REGION
    pltpu.sync_copy(counts.at[pl.ds(wid * 16, 16)], cbuf)
    n = jnp.max(cbuf[...])
    nchunks = (n + _CHUNK - 1) // _CHUNK

    def chunk(c, carry):
        base = rbase + c * _CHUNK
        pltpu.sync_copy(jlist.at[pl.ds(base, _CHUNK)], jbuf)
        pltpu.sync_copy(vals2d.at[pl.ds(base, _CHUNK)], rows)
        pltpu.async_copy(rows, out.at[jbuf], sem).wait()
        return carry
    lax.fori_loop(0, nchunks, chunk, 0)


_CHUNK = 1024


@jax.jit
def _lookup(embedding, inputs):
    mesh = plsc.VectorSubcoreMesh(core_axis_name="c", subcore_axis_name="s")
    k1 = pl.kernel(
        _k1_body,
        mesh=mesh,
        out_type=(
            jax.ShapeDtypeStruct((_NW * _REGION * D,), jnp.float32),
            jax.ShapeDtypeStruct((_NW * _REGION,), jnp.int32),
            jax.ShapeDtypeStruct((_NW * 16,), jnp.int32),
        ),
        scratch_types=[
            pltpu.VMEM((_REGION,), jnp.int32),
            pltpu.VMEM((_REGION,), jnp.int32),
            pltpu.VMEM((_REGION,), jnp.int32),
            pltpu.VMEM((_REGION,), jnp.int32),
            pltpu.VMEM((D, _GCOLS), jnp.float32),
            pltpu.VMEM((D, _GCOLS), jnp.float32),
            pltpu.VMEM((D, NODE - _TAIL_COL), jnp.float32),
            pltpu.VMEM((16 * D,), jnp.float32),
            pltpu.VMEM((16,), jnp.int32),
            pltpu.SemaphoreType.DMA,
            pltpu.SemaphoreType.DMA,
        ],
        compiler_params=pltpu.CompilerParams(
            use_tc_tiling_on_sc=True, needs_layout_passes=False),
    )
    vals, jlist, counts = k1(embedding.T, inputs)

    k2 = pl.kernel(
        _k2_body,
        mesh=mesh,
        out_type=jax.ShapeDtypeStruct((_OUT_ROWS, D), jnp.float32),
        scratch_types=[
            pltpu.VMEM((_CHUNK,), jnp.int32),
            pltpu.VMEM((_CHUNK, D), jnp.float32),
            pltpu.VMEM((16,), jnp.int32),
            pltpu.SemaphoreType.DMA,
        ],
        compiler_params=pltpu.CompilerParams(
            use_tc_tiling_on_sc=False, needs_layout_passes=False),
    )
    out = k2(vals.reshape(_NW * _REGION, D), jlist, counts)
    return out[:B]


def kernel(inputs, embedding):
    return _lookup(embedding, inputs.astype(jnp.int32))
